# baseline (speedup 1.0000x reference)
import jax
import jax.numpy as jnp
from jax.experimental import pallas as pl


def _one_hot_kernel(y_ref, o_ref):
    n, e = o_ref.shape
    classes = jax.lax.broadcasted_iota(jnp.int32, (n, e), 1)
    o_ref[:] = (y_ref[:] == classes).astype(jnp.float32)


def kernel(functional_samples, x, expected_logbeta, y, mollify, mixer, temperature):
    num_classes = functional_samples.shape[1]
    n = y.shape[0]
    y32 = y.astype(jnp.int32).reshape(n, 1)
    out = pl.pallas_call(
        _one_hot_kernel,
        out_shape=jax.ShapeDtypeStruct((n, num_classes), jnp.float32),
    )(y32)
    zero = jnp.zeros((), dtype=jnp.float32)
    return (out, zero, out)
